# Initial kernel scaffold; baseline (speedup 1.0000x reference)
import jax
import jax.numpy as jnp
from jax.experimental import pallas as pl


def kernel(color_indices, table, W, b):
    raise NotImplementedError("write your pallas kernel here")


# speedup vs baseline: 3.2461x; 3.2461x over previous
import functools

import jax
import jax.numpy as jnp
from jax import lax
from jax.experimental import pallas as pl
from jax.experimental.pallas import tpu as pltpu
from jax.experimental.pallas import tpu_sc as plsc


def _proj_body(t_ref, wt_ref, b_ref, o_ref):
    o_ref[...] = (
        jnp.dot(t_ref[...], wt_ref[...], preferred_element_type=jnp.float32)
        + b_ref[...]
    )


def _project_table(table, Wt, b2):
    V, D = table.shape
    BLK = 4000
    assert V % BLK == 0
    return pl.pallas_call(
        _proj_body,
        grid=(V // BLK,),
        in_specs=[
            pl.BlockSpec((BLK, D), lambda i: (i, 0)),
            pl.BlockSpec((D, D), lambda i: (0, 0)),
            pl.BlockSpec((1, D), lambda i: (0, 0)),
        ],
        out_specs=pl.BlockSpec((BLK, D), lambda i: (i, 0)),
        out_shape=jax.ShapeDtypeStruct((V, D), jnp.float32),
    )(table, Wt, b2)


_BLK = 128


def _make_gather(V, D, NW, NC, n_blk):
    mesh = plsc.VectorSubcoreMesh(core_axis_name="c", subcore_axis_name="s")

    @functools.partial(
        pl.kernel,
        mesh=mesh,
        out_type=jax.ShapeDtypeStruct((NW * n_blk * _BLK, D), jnp.float32),
        scratch_types=[
            pltpu.VMEM((n_blk, _BLK), jnp.int32),
            pltpu.VMEM((_BLK, D), jnp.float32),
            pltpu.SemaphoreType.DMA,
        ],
        compiler_params=pltpu.CompilerParams(use_tc_tiling_on_sc=False),
    )
    def gather(tab_hbm, idx_hbm, out_hbm, idx_v, rows_v, sem):
        wid = lax.axis_index("s") * NC + lax.axis_index("c")
        pltpu.sync_copy(idx_hbm.at[wid], idx_v)
        base_row = wid * (n_blk * _BLK)

        def loop(j, carry):
            pltpu.async_copy(tab_hbm.at[idx_v.at[j]], rows_v, sem).wait()
            pltpu.sync_copy(rows_v, out_hbm.at[pl.ds(base_row + j * _BLK, _BLK)])
            return carry

        lax.fori_loop(0, n_blk, loop, 0)

    return gather


def kernel(color_indices, table, W, b):
    B, L = color_indices.shape
    V, D = table.shape
    info = plsc.get_sparse_core_info()
    NC, NS = info.num_cores, info.num_subcores
    NW = NC * NS
    total = B * L
    assert total % (NW * _BLK) == 0
    n_blk = total // (NW * _BLK)

    proj = _project_table(table, W.T, b.reshape(1, D))
    idx = color_indices.astype(jnp.int32).reshape(NW, n_blk, _BLK)
    out = _make_gather(V, D, NW, NC, n_blk)(proj, idx)
    return out.reshape(B, L, D)


# speedup vs baseline: 3.5526x; 1.0944x over previous
import functools

import jax
import jax.numpy as jnp
from jax import lax
from jax.experimental import pallas as pl
from jax.experimental.pallas import tpu as pltpu
from jax.experimental.pallas import tpu_sc as plsc


def _proj_body(t_ref, wt_ref, b_ref, o_ref):
    o_ref[...] = (
        jnp.dot(t_ref[...], wt_ref[...], preferred_element_type=jnp.float32)
        + b_ref[...]
    )


def _project_table(table, W, b):
    V, D = table.shape
    t2 = table.reshape(V // 2, 2 * D)
    Wt = W.T
    W2 = jnp.zeros((2 * D, 2 * D), jnp.float32)
    W2 = W2.at[:D, :D].set(Wt).at[D:, D:].set(Wt)
    b2 = jnp.concatenate([b, b]).reshape(1, 2 * D)
    BLK = 2000
    assert (V // 2) % BLK == 0
    proj2 = pl.pallas_call(
        _proj_body,
        grid=(V // 2 // BLK,),
        in_specs=[
            pl.BlockSpec((BLK, 2 * D), lambda i: (i, 0)),
            pl.BlockSpec((2 * D, 2 * D), lambda i: (0, 0)),
            pl.BlockSpec((1, 2 * D), lambda i: (0, 0)),
        ],
        out_specs=pl.BlockSpec((BLK, 2 * D), lambda i: (i, 0)),
        out_shape=jax.ShapeDtypeStruct((V // 2, 2 * D), jnp.float32),
    )(t2, W2, b2)
    return proj2.reshape(V, D)


_BLK = 128


def _make_gather(V, D, NW, NC, n_blk):
    mesh = plsc.VectorSubcoreMesh(core_axis_name="c", subcore_axis_name="s")

    @functools.partial(
        pl.kernel,
        mesh=mesh,
        out_type=jax.ShapeDtypeStruct((NW * n_blk * _BLK, D), jnp.float32),
        scratch_types=[
            pltpu.VMEM((n_blk, _BLK), jnp.int32),
            pltpu.VMEM((_BLK, D), jnp.float32),
            pltpu.SemaphoreType.DMA,
        ],
        compiler_params=pltpu.CompilerParams(use_tc_tiling_on_sc=False),
    )
    def gather(tab_hbm, idx_hbm, out_hbm, idx_v, rows_v, sem):
        wid = lax.axis_index("s") * NC + lax.axis_index("c")
        pltpu.sync_copy(idx_hbm.at[wid], idx_v)
        base_row = wid * (n_blk * _BLK)

        def loop(j, carry):
            pltpu.async_copy(tab_hbm.at[idx_v.at[j]], rows_v, sem).wait()
            pltpu.sync_copy(rows_v, out_hbm.at[pl.ds(base_row + j * _BLK, _BLK)])
            return carry

        lax.fori_loop(0, n_blk, loop, 0)

    return gather


def kernel(color_indices, table, W, b):
    B, L = color_indices.shape
    V, D = table.shape
    info = plsc.get_sparse_core_info()
    NC, NS = info.num_cores, info.num_subcores
    NW = NC * NS
    total = B * L
    assert total % (NW * _BLK) == 0
    n_blk = total // (NW * _BLK)

    proj = _project_table(table, W, b)
    idx = color_indices.astype(jnp.int32).reshape(NW, n_blk, _BLK)
    out = _make_gather(V, D, NW, NC, n_blk)(proj, idx)
    return out.reshape(B, L, D)
